# Initial kernel scaffold; baseline (speedup 1.0000x reference)
import jax
import jax.numpy as jnp
from jax.experimental import pallas as pl


def kernel(inp, idx, src):
    raise NotImplementedError("write your pallas kernel here")


# speedup vs baseline: 36.9191x; 36.9191x over previous
import functools

import jax
import jax.numpy as jnp
from jax import lax
from jax.experimental import pallas as pl
from jax.experimental.pallas import tpu as pltpu
from jax.experimental.pallas import tpu_sc as plsc

_NW = 32
_CH = 8192
_UNROLL = 8


def _sc_scatter(idx_t, src_t, m_rows):
    d_cols, n_upd = idx_t.shape
    half = m_rows // 2
    tasks = d_cols * 2
    tasks_per_worker = tasks // _NW
    n_chunks = n_upd // _CH

    mesh = plsc.VectorSubcoreMesh(core_axis_name="c", subcore_axis_name="s")

    @functools.partial(
        pl.kernel,
        out_type=jax.ShapeDtypeStruct((d_cols, m_rows), jnp.float32),
        mesh=mesh,
        scratch_types=[
            pltpu.VMEM((half,), jnp.float32),
            pltpu.VMEM((_CH,), jnp.int32),
            pltpu.VMEM((_CH,), jnp.float32),
        ],
        compiler_params=pltpu.CompilerParams(needs_layout_passes=False),
    )
    def scatter_kernel(idx_hbm, src_hbm, delta_hbm, acc, ibuf, sbuf):
        wid = lax.axis_index("s") * 2 + lax.axis_index("c")
        zeros16 = jnp.zeros((16,), jnp.float32)

        def task_body(t, carry):
            task = t * _NW + wid
            col = task // 2
            lo = (task % 2) * half

            def zero_body(i, c2):
                for u in range(_UNROLL):
                    acc[pl.ds(i * 16 * _UNROLL + u * 16, 16)] = zeros16
                return c2

            lax.fori_loop(0, half // (16 * _UNROLL), zero_body, 0)

            def chunk_body(ch, c2):
                pltpu.sync_copy(idx_hbm.at[col, pl.ds(ch * _CH, _CH)], ibuf)
                pltpu.sync_copy(src_hbm.at[col, pl.ds(ch * _CH, _CH)], sbuf)

                def vec_body(k, c3):
                    for u in range(_UNROLL):
                        off = k * 16 * _UNROLL + u * 16
                        iv = ibuf[pl.ds(off, 16)]
                        sv = sbuf[pl.ds(off, 16)]
                        local = iv - lo
                        msk = (iv >= lo) & (local < half)
                        plsc.addupdate_scatter(acc, [local], sv, mask=msk)
                    return c3

                lax.fori_loop(0, _CH // (16 * _UNROLL), vec_body, 0)
                return c2

            lax.fori_loop(0, n_chunks, chunk_body, 0)
            pltpu.sync_copy(acc, delta_hbm.at[col, pl.ds(lo, half)])
            return carry

        lax.fori_loop(0, tasks_per_worker, task_body, 0)

    return scatter_kernel(idx_t, src_t)


def _combine(inp, delta_t):
    m_rows, d_cols = inp.shape
    bm = 512

    def body(inp_ref, dt_ref, out_ref):
        out_ref[...] = inp_ref[...] + dt_ref[...].T

    return pl.pallas_call(
        body,
        grid=(m_rows // bm,),
        in_specs=[
            pl.BlockSpec((bm, d_cols), lambda i: (i, 0)),
            pl.BlockSpec((d_cols, bm), lambda i: (0, i)),
        ],
        out_specs=pl.BlockSpec((bm, d_cols), lambda i: (i, 0)),
        out_shape=jax.ShapeDtypeStruct((m_rows, d_cols), jnp.float32),
    )(inp, delta_t)


def kernel(inp, idx, src):
    m_rows, _ = inp.shape
    idx_t = idx.astype(jnp.int32).T
    src_t = src.T
    delta_t = _sc_scatter(idx_t, src_t, m_rows)
    return _combine(inp, delta_t)


# speedup vs baseline: 49.7672x; 1.3480x over previous
import functools

import jax
import jax.numpy as jnp
from jax import lax
from jax.experimental import pallas as pl
from jax.experimental.pallas import tpu as pltpu
from jax.experimental.pallas import tpu_sc as plsc

_NW = 32
_CH = 8192
_UNROLL = 8


def _sc_scatter(idx_t, src_t, m_rows):
    d_cols, n_upd = idx_t.shape
    half = m_rows // 2
    cols_per_worker = d_cols // _NW
    n_chunks = n_upd // _CH

    mesh = plsc.VectorSubcoreMesh(core_axis_name="c", subcore_axis_name="s")

    @functools.partial(
        pl.kernel,
        out_type=jax.ShapeDtypeStruct((d_cols, m_rows), jnp.float32),
        mesh=mesh,
        scratch_types=[
            pltpu.VMEM((half,), jnp.float32),
            pltpu.VMEM((_CH,), jnp.int32),
            pltpu.VMEM((_CH,), jnp.float32),
            pltpu.VMEM((_CH,), jnp.int32),
            pltpu.VMEM((_CH,), jnp.float32),
            pltpu.SemaphoreType.DMA,
            pltpu.SemaphoreType.DMA,
        ],
        compiler_params=pltpu.CompilerParams(needs_layout_passes=False),
    )
    def scatter_kernel(idx_hbm, src_hbm, delta_hbm,
                       acc, ibuf0, sbuf0, ibuf1, sbuf1, sem0, sem1):
        wid = lax.axis_index("s") * 2 + lax.axis_index("c")
        zeros16 = jnp.zeros((16,), jnp.float32)
        bufs = ((ibuf0, sbuf0, sem0), (ibuf1, sbuf1, sem1))

        def run_half(is_high):
            lo = half if is_high else 0

            def task_body(t, carry):
                col = t * _NW + wid

                def fire(ch, b):
                    ib, sb, sem = bufs[b]
                    pltpu.async_copy(
                        idx_hbm.at[col, pl.ds(ch * _CH, _CH)], ib, sem)
                    pltpu.async_copy(
                        src_hbm.at[col, pl.ds(ch * _CH, _CH)], sb, sem)

                def drain(ch, b):
                    ib, sb, sem = bufs[b]
                    pltpu.make_async_copy(
                        idx_hbm.at[col, pl.ds(ch * _CH, _CH)], ib, sem).wait()
                    pltpu.make_async_copy(
                        src_hbm.at[col, pl.ds(ch * _CH, _CH)], sb, sem).wait()

                fire(0, 0)

                def zero_body(i, c2):
                    for u in range(_UNROLL):
                        acc[pl.ds(i * 16 * _UNROLL + u * 16, 16)] = zeros16
                    return c2

                lax.fori_loop(0, half // (16 * _UNROLL), zero_body, 0)

                for ch in range(n_chunks):
                    b = ch % 2
                    if ch + 1 < n_chunks:
                        fire(ch + 1, 1 - b)
                    drain(ch, b)
                    ib, sb, _ = bufs[b]

                    def vec_body(k, c3, ib=ib, sb=sb):
                        for u in range(_UNROLL):
                            off = k * 16 * _UNROLL + u * 16
                            iv = ib[pl.ds(off, 16)]
                            sv = sb[pl.ds(off, 16)]
                            if is_high:
                                local = iv - half
                                msk = iv >= half
                            else:
                                local = iv
                                msk = iv < half
                            plsc.addupdate_scatter(acc, [local], sv, mask=msk)
                        return c3

                    lax.fori_loop(0, _CH // (16 * _UNROLL), vec_body, 0)

                pltpu.sync_copy(acc, delta_hbm.at[col, pl.ds(lo, half)])
                return carry

            lax.fori_loop(0, cols_per_worker, task_body, 0)

        run_half(False)
        run_half(True)

    return scatter_kernel(idx_t, src_t)


def _combine(inp, delta_t):
    m_rows, d_cols = inp.shape
    bm = 512

    def body(inp_ref, dt_ref, out_ref):
        out_ref[...] = inp_ref[...] + dt_ref[...].T

    return pl.pallas_call(
        body,
        grid=(m_rows // bm,),
        in_specs=[
            pl.BlockSpec((bm, d_cols), lambda i: (i, 0)),
            pl.BlockSpec((d_cols, bm), lambda i: (0, i)),
        ],
        out_specs=pl.BlockSpec((bm, d_cols), lambda i: (i, 0)),
        out_shape=jax.ShapeDtypeStruct((m_rows, d_cols), jnp.float32),
    )(inp, delta_t)


def kernel(inp, idx, src):
    m_rows, _ = inp.shape
    idx_t = idx.astype(jnp.int32).T
    src_t = src.T
    delta_t = _sc_scatter(idx_t, src_t, m_rows)
    return _combine(inp, delta_t)


# speedup vs baseline: 77.4695x; 1.5566x over previous
import functools

import jax
import jax.numpy as jnp
from jax import lax
from jax.experimental import pallas as pl
from jax.experimental.pallas import tpu as pltpu
from jax.experimental.pallas import tpu_sc as plsc

_NW = 32
_CH = 8192
_UNROLL = 8


def _sc_scatter(idx_t, src_t, m_rows):
    d_cols, n_upd = idx_t.shape
    half = m_rows // 2
    cols_per_worker = d_cols // _NW
    n_chunks = n_upd // _CH

    mesh = plsc.VectorSubcoreMesh(core_axis_name="c", subcore_axis_name="s")

    @functools.partial(
        pl.kernel,
        out_type=jax.ShapeDtypeStruct((d_cols, m_rows), jnp.float32),
        mesh=mesh,
        scratch_types=[
            pltpu.VMEM((half,), jnp.float32),
            pltpu.VMEM((_CH,), jnp.int32),
            pltpu.VMEM((_CH,), jnp.float32),
            pltpu.VMEM((_CH,), jnp.int32),
            pltpu.VMEM((_CH,), jnp.float32),
            pltpu.SemaphoreType.DMA,
            pltpu.SemaphoreType.DMA,
        ],
        compiler_params=pltpu.CompilerParams(needs_layout_passes=False),
    )
    def scatter_kernel(idx_hbm, src_hbm, delta_hbm,
                       acc, ibuf0, sbuf0, ibuf1, sbuf1, sem0, sem1):
        wid = lax.axis_index("s") * 2 + lax.axis_index("c")
        zeros16 = jnp.zeros((16,), jnp.float32)
        bufs = ((ibuf0, sbuf0, sem0), (ibuf1, sbuf1, sem1))

        def run_half(is_high):
            lo = half if is_high else 0

            def task_body(t, carry):
                col = t * _NW + wid

                def fire(ch, b):
                    ib, sb, sem = bufs[b]
                    pltpu.async_copy(
                        idx_hbm.at[col, pl.ds(ch * _CH, _CH)], ib, sem)
                    pltpu.async_copy(
                        src_hbm.at[col, pl.ds(ch * _CH, _CH)], sb, sem)

                def drain(ch, b):
                    ib, sb, sem = bufs[b]
                    pltpu.make_async_copy(
                        idx_hbm.at[col, pl.ds(ch * _CH, _CH)], ib, sem).wait()
                    pltpu.make_async_copy(
                        src_hbm.at[col, pl.ds(ch * _CH, _CH)], sb, sem).wait()

                fire(0, 0)

                @plsc.parallel_loop(0, half // 16, 1, unroll=_UNROLL)
                def zero_body(i):
                    acc[pl.ds(i * 16, 16)] = zeros16

                for ch in range(n_chunks):
                    b = ch % 2
                    if ch + 1 < n_chunks:
                        fire(ch + 1, 1 - b)
                    drain(ch, b)
                    ib, sb, _ = bufs[b]

                    @plsc.parallel_loop(0, _CH // 16, 1, unroll=_UNROLL)
                    def vec_body(k, ib=ib, sb=sb):
                        iv = ib[pl.ds(k * 16, 16)]
                        sv = sb[pl.ds(k * 16, 16)]
                        if is_high:
                            local = iv - half
                            msk = iv >= half
                        else:
                            local = iv
                            msk = iv < half
                        plsc.addupdate_scatter(acc, [local], sv, mask=msk)

                pltpu.sync_copy(acc, delta_hbm.at[col, pl.ds(lo, half)])
                return carry

            lax.fori_loop(0, cols_per_worker, task_body, 0)

        run_half(False)
        run_half(True)

    return scatter_kernel(idx_t, src_t)


def _combine(inp, delta_t):
    m_rows, d_cols = inp.shape
    bm = 512

    def body(inp_ref, dt_ref, out_ref):
        out_ref[...] = inp_ref[...] + dt_ref[...].T

    return pl.pallas_call(
        body,
        grid=(m_rows // bm,),
        in_specs=[
            pl.BlockSpec((bm, d_cols), lambda i: (i, 0)),
            pl.BlockSpec((d_cols, bm), lambda i: (0, i)),
        ],
        out_specs=pl.BlockSpec((bm, d_cols), lambda i: (i, 0)),
        out_shape=jax.ShapeDtypeStruct((m_rows, d_cols), jnp.float32),
    )(inp, delta_t)


def kernel(inp, idx, src):
    m_rows, _ = inp.shape
    idx_t = idx.astype(jnp.int32).T
    src_t = src.T
    delta_t = _sc_scatter(idx_t, src_t, m_rows)
    return _combine(inp, delta_t)


# speedup vs baseline: 87.1358x; 1.1248x over previous
import functools

import jax
import jax.numpy as jnp
from jax import lax
from jax.experimental import pallas as pl
from jax.experimental.pallas import tpu as pltpu
from jax.experimental.pallas import tpu_sc as plsc

_NW = 32
_CH = 8192
_UNROLL = 8


def _sc_scatter(idx_t, src_t, m_rows):
    d_cols, n_upd = idx_t.shape
    half = m_rows // 2
    cols_per_worker = d_cols // _NW
    n_chunks = n_upd // _CH

    mesh = plsc.VectorSubcoreMesh(core_axis_name="c", subcore_axis_name="s")

    @functools.partial(
        pl.kernel,
        out_type=jax.ShapeDtypeStruct((d_cols, m_rows), jnp.float32),
        mesh=mesh,
        scratch_types=[
            pltpu.VMEM((half,), jnp.float32),
            pltpu.VMEM((_CH,), jnp.int32),
            pltpu.VMEM((_CH,), jnp.float32),
            pltpu.VMEM((_CH,), jnp.int32),
            pltpu.VMEM((_CH,), jnp.float32),
            pltpu.SemaphoreType.DMA,
            pltpu.SemaphoreType.DMA,
        ],
        compiler_params=pltpu.CompilerParams(needs_layout_passes=False),
    )
    def scatter_kernel(idx_hbm, src_hbm, delta_hbm,
                       acc, ibuf0, sbuf0, ibuf1, sbuf1, sem0, sem1):
        wid = lax.axis_index("s") * 2 + lax.axis_index("c")
        zeros16 = jnp.zeros((16,), jnp.float32)
        bufs = ((ibuf0, sbuf0, sem0), (ibuf1, sbuf1, sem1))

        def run_half(is_high):
            lo = half if is_high else 0

            def task_body(t, carry):
                col = t * _NW + wid

                def fire(ch, b):
                    ib, sb, sem = bufs[b]
                    pltpu.async_copy(
                        idx_hbm.at[col, pl.ds(ch * _CH, _CH)], ib, sem)
                    pltpu.async_copy(
                        src_hbm.at[col, pl.ds(ch * _CH, _CH)], sb, sem)

                def drain(ch, b):
                    ib, sb, sem = bufs[b]
                    pltpu.make_async_copy(
                        idx_hbm.at[col, pl.ds(ch * _CH, _CH)], ib, sem).wait()
                    pltpu.make_async_copy(
                        src_hbm.at[col, pl.ds(ch * _CH, _CH)], sb, sem).wait()

                fire(0, 0)

                @plsc.parallel_loop(0, half // 16, 1, unroll=_UNROLL)
                def zero_body(i):
                    acc[pl.ds(i * 16, 16)] = zeros16

                for ch in range(n_chunks):
                    b = ch % 2
                    if ch + 1 < n_chunks:
                        fire(ch + 1, 1 - b)
                    drain(ch, b)
                    ib, sb, _ = bufs[b]

                    @plsc.parallel_loop(0, _CH // 16, 1, unroll=_UNROLL)
                    def vec_body(k, ib=ib, sb=sb):
                        iv = ib[pl.ds(k * 16, 16)]
                        sv = sb[pl.ds(k * 16, 16)]
                        if is_high:
                            local = iv - half
                            msk = iv >= half
                        else:
                            local = iv
                            msk = iv < half
                        plsc.addupdate_scatter(acc, [local], sv, mask=msk)

                pltpu.sync_copy(acc, delta_hbm.at[col, pl.ds(lo, half)])
                return carry

            lax.fori_loop(0, cols_per_worker, task_body, 0)

        run_half(False)
        run_half(True)

    return scatter_kernel(idx_t, src_t)


def _combine(inp, delta_t):
    m_rows, d_cols = inp.shape
    bm = 1024

    def body(inp_ref, dt_ref, out_ref):
        out_ref[...] = inp_ref[...] + dt_ref[...].T

    return pl.pallas_call(
        body,
        grid=(m_rows // bm,),
        in_specs=[
            pl.BlockSpec((bm, d_cols), lambda i: (i, 0)),
            pl.BlockSpec((d_cols, bm), lambda i: (0, i)),
        ],
        out_specs=pl.BlockSpec((bm, d_cols), lambda i: (i, 0)),
        out_shape=jax.ShapeDtypeStruct((m_rows, d_cols), jnp.float32),
    )(inp, delta_t)


def kernel(inp, idx, src):
    m_rows, _ = inp.shape
    idx_t = idx.astype(jnp.int32).T
    src_t = src.T
    delta_t = _sc_scatter(idx_t, src_t, m_rows)
    return _combine(inp, delta_t)


# speedup vs baseline: 92.5429x; 1.0621x over previous
import functools

import jax
import jax.numpy as jnp
from jax import lax
from jax.experimental import pallas as pl
from jax.experimental.pallas import tpu as pltpu
from jax.experimental.pallas import tpu_sc as plsc

_NW = 32
_CH = 8192
_UNROLL = 8


def _sc_scatter(idx_t, src_t, m_rows):
    d_cols, n_upd = idx_t.shape
    half = m_rows // 2
    cols_per_worker = d_cols // _NW
    n_chunks = n_upd // _CH

    mesh = plsc.VectorSubcoreMesh(core_axis_name="c", subcore_axis_name="s")

    @functools.partial(
        pl.kernel,
        out_type=jax.ShapeDtypeStruct((d_cols, m_rows), jnp.float32),
        mesh=mesh,
        scratch_types=[
            pltpu.VMEM((half,), jnp.float32),
            pltpu.VMEM((_CH,), jnp.int32),
            pltpu.VMEM((_CH,), jnp.float32),
            pltpu.VMEM((_CH,), jnp.int32),
            pltpu.VMEM((_CH,), jnp.float32),
            pltpu.SemaphoreType.DMA,
            pltpu.SemaphoreType.DMA,
        ],
        compiler_params=pltpu.CompilerParams(needs_layout_passes=False),
    )
    def scatter_kernel(idx_hbm, src_hbm, delta_hbm,
                       acc, ibuf0, sbuf0, ibuf1, sbuf1, sem0, sem1):
        wid = lax.axis_index("s") * 2 + lax.axis_index("c")
        zeros16 = jnp.zeros((16,), jnp.float32)
        bufs = ((ibuf0, sbuf0, sem0), (ibuf1, sbuf1, sem1))

        def run_half(is_high):
            lo = half if is_high else 0

            def task_body(t, carry):
                col = t * _NW + wid

                def fire(ch, b):
                    ib, sb, sem = bufs[b]
                    pltpu.async_copy(
                        idx_hbm.at[col, pl.ds(ch * _CH, _CH)], ib, sem)
                    pltpu.async_copy(
                        src_hbm.at[col, pl.ds(ch * _CH, _CH)], sb, sem)

                def drain(ch, b):
                    ib, sb, sem = bufs[b]
                    pltpu.make_async_copy(
                        idx_hbm.at[col, pl.ds(ch * _CH, _CH)], ib, sem).wait()
                    pltpu.make_async_copy(
                        src_hbm.at[col, pl.ds(ch * _CH, _CH)], sb, sem).wait()

                fire(0, 0)

                @plsc.parallel_loop(0, half // 16, 1, unroll=_UNROLL)
                def zero_body(i):
                    acc[pl.ds(i * 16, 16)] = zeros16

                for ch in range(n_chunks):
                    b = ch % 2
                    if ch + 1 < n_chunks:
                        fire(ch + 1, 1 - b)
                    drain(ch, b)
                    ib, sb, _ = bufs[b]

                    @plsc.parallel_loop(0, _CH // 16, 1, unroll=_UNROLL)
                    def vec_body(k, ib=ib, sb=sb):
                        iv = ib[pl.ds(k * 16, 16)]
                        sv = sb[pl.ds(k * 16, 16)]
                        if is_high:
                            local = iv - half
                            msk = iv >= half
                        else:
                            local = iv
                            msk = iv < half
                        plsc.addupdate_scatter(acc, [local], sv, mask=msk)

                pltpu.sync_copy(acc, delta_hbm.at[col, pl.ds(lo, half)])
                return carry

            lax.fori_loop(0, cols_per_worker, task_body, 0)

        run_half(False)
        run_half(True)

    return scatter_kernel(idx_t, src_t)


def _combine(inp, delta_t):
    m_rows, d_cols = inp.shape
    bm = 2048

    def body(inp_ref, dt_ref, out_ref):
        out_ref[...] = inp_ref[...] + dt_ref[...].T

    return pl.pallas_call(
        body,
        grid=(m_rows // bm,),
        in_specs=[
            pl.BlockSpec((bm, d_cols), lambda i: (i, 0)),
            pl.BlockSpec((d_cols, bm), lambda i: (0, i)),
        ],
        out_specs=pl.BlockSpec((bm, d_cols), lambda i: (i, 0)),
        out_shape=jax.ShapeDtypeStruct((m_rows, d_cols), jnp.float32),
    )(inp, delta_t)


def kernel(inp, idx, src):
    m_rows, _ = inp.shape
    idx_t = idx.astype(jnp.int32).T
    src_t = src.T
    delta_t = _sc_scatter(idx_t, src_t, m_rows)
    return _combine(inp, delta_t)


# speedup vs baseline: 93.7067x; 1.0126x over previous
import functools

import jax
import jax.numpy as jnp
from jax import lax
from jax.experimental import pallas as pl
from jax.experimental.pallas import tpu as pltpu
from jax.experimental.pallas import tpu_sc as plsc

_NW = 32
_CH = 8192
_UNROLL = 8


def _sc_scatter(idx_t, src_t, m_rows):
    d_cols, n_upd = idx_t.shape
    half = m_rows // 2
    cols_per_worker = d_cols // _NW
    n_chunks = n_upd // _CH

    mesh = plsc.VectorSubcoreMesh(core_axis_name="c", subcore_axis_name="s")

    @functools.partial(
        pl.kernel,
        out_type=jax.ShapeDtypeStruct((d_cols, m_rows), jnp.float32),
        mesh=mesh,
        scratch_types=[
            pltpu.VMEM((half,), jnp.float32),
            pltpu.VMEM((_CH,), jnp.int32),
            pltpu.VMEM((_CH,), jnp.float32),
            pltpu.VMEM((_CH,), jnp.int32),
            pltpu.VMEM((_CH,), jnp.float32),
            pltpu.SemaphoreType.DMA,
            pltpu.SemaphoreType.DMA,
        ],
        compiler_params=pltpu.CompilerParams(needs_layout_passes=False),
    )
    def scatter_kernel(idx_hbm, src_hbm, delta_hbm,
                       acc, ibuf0, sbuf0, ibuf1, sbuf1, sem0, sem1):
        wid = lax.axis_index("s") * 2 + lax.axis_index("c")
        zeros16 = jnp.zeros((16,), jnp.float32)
        bufs = ((ibuf0, sbuf0, sem0), (ibuf1, sbuf1, sem1))

        def run_half(is_high):
            lo = half if is_high else 0

            def task_body(t, carry):
                col = t * _NW + wid

                def fire(ch, b):
                    ib, sb, sem = bufs[b]
                    pltpu.async_copy(
                        idx_hbm.at[col, pl.ds(ch * _CH, _CH)], ib, sem)
                    pltpu.async_copy(
                        src_hbm.at[col, pl.ds(ch * _CH, _CH)], sb, sem)

                def drain(ch, b):
                    ib, sb, sem = bufs[b]
                    pltpu.make_async_copy(
                        idx_hbm.at[col, pl.ds(ch * _CH, _CH)], ib, sem).wait()
                    pltpu.make_async_copy(
                        src_hbm.at[col, pl.ds(ch * _CH, _CH)], sb, sem).wait()

                fire(0, 0)

                @plsc.parallel_loop(0, half // 16, 1, unroll=_UNROLL)
                def zero_body(i):
                    acc[pl.ds(i * 16, 16)] = zeros16

                for ch in range(n_chunks):
                    b = ch % 2
                    if ch + 1 < n_chunks:
                        fire(ch + 1, 1 - b)
                    drain(ch, b)
                    ib, sb, _ = bufs[b]

                    @plsc.parallel_loop(0, _CH // 16, 1, unroll=_UNROLL)
                    def vec_body(k, ib=ib, sb=sb):
                        iv = ib[pl.ds(k * 16, 16)]
                        sv = sb[pl.ds(k * 16, 16)]
                        if is_high:
                            local = iv - half
                            msk = iv >= half
                        else:
                            local = iv
                            msk = iv < half
                        plsc.addupdate_scatter(acc, [local], sv, mask=msk)

                pltpu.sync_copy(acc, delta_hbm.at[col, pl.ds(lo, half)])
                return carry

            lax.fori_loop(0, cols_per_worker, task_body, 0)

        run_half(False)
        run_half(True)

    return scatter_kernel(idx_t, src_t)


def _combine(inp, delta_t):
    m_rows, d_cols = inp.shape
    bm = 4096

    def body(inp_ref, dt_ref, out_ref):
        out_ref[...] = inp_ref[...] + dt_ref[...].T

    return pl.pallas_call(
        body,
        grid=(m_rows // bm,),
        in_specs=[
            pl.BlockSpec((bm, d_cols), lambda i: (i, 0)),
            pl.BlockSpec((d_cols, bm), lambda i: (0, i)),
        ],
        out_specs=pl.BlockSpec((bm, d_cols), lambda i: (i, 0)),
        out_shape=jax.ShapeDtypeStruct((m_rows, d_cols), jnp.float32),
    )(inp, delta_t)


def kernel(inp, idx, src):
    m_rows, _ = inp.shape
    idx_t = idx.astype(jnp.int32).T
    src_t = src.T
    delta_t = _sc_scatter(idx_t, src_t, m_rows)
    return _combine(inp, delta_t)
